# Initial kernel scaffold; baseline (speedup 1.0000x reference)
import jax
import jax.numpy as jnp
from jax.experimental import pallas as pl


def kernel(x, edge_index, W_l1, b_l1, W_r1, W_l2, b_l2, W_r2):
    raise NotImplementedError("write your pallas kernel here")


# speedup vs baseline: 3.7943x; 3.7943x over previous
import functools

import jax
import jax.numpy as jnp
from jax import lax
from jax.experimental import pallas as pl
from jax.experimental.pallas import tpu as pltpu
from jax.experimental.pallas import tpu_sc as plsc

N = 10000
NPAD = 10240
E = 160000
D = 256
DH = 128
EB = 80
TILES = 16
NW = 32
EPT = E // TILES
NB = EPT // EB
RPT = NPAD // TILES
RB = 1024
GRID = NPAD // RB


def _seg_sum_body(count, *refs):
    if count:
        (x0, x1, srch, dsth, agg0, agg1, cnt_out,
         src_v, dst_v, rows_v, hist_v, acc, sem) = refs
    else:
        (x0, x1, srch, dsth, agg0, agg1,
         src_v, dst_v, rows_v, hist_v, acc, sem) = refs
        cnt_out = None
    c = lax.axis_index("c")
    t = lax.axis_index("s")
    r0 = t * RPT

    def _zero_rows(k, _):
        i = k // (DH // 16)
        j = k % (DH // 16)
        rows_v[i, pl.ds(j * 16, 16)] = jnp.zeros((16,), jnp.float32)
        return 0
    lax.fori_loop(0, EB * (DH // 16), _zero_rows, 0)

    def _zero_acc(k, _):
        pltpu.sync_copy(rows_v, acc.at[pl.ds(r0 + k * EB, EB)])
        return 0
    lax.fori_loop(0, RPT // EB, _zero_acc, 0)

    if count:
        def _zero_hist(k, _):
            hist_v[pl.ds(k * 16, 16)] = jnp.zeros((16,), jnp.float32)
            return 0
        lax.fori_loop(0, NPAD // 16, _zero_hist, 0)

    plsc.subcore_barrier()

    ebase = t * EPT

    def _step(k, _):
        off = ebase + k * EB
        pltpu.sync_copy(srch.at[pl.ds(off, EB)], src_v)
        pltpu.sync_copy(dsth.at[pl.ds(off, EB)], dst_v)

        @pl.when(c == 0)
        def _():
            pltpu.async_copy(x0.at[src_v], rows_v, sem).wait()

        @pl.when(c == 1)
        def _():
            pltpu.async_copy(x1.at[src_v], rows_v, sem).wait()

        pltpu.sync_copy(rows_v, acc.at[dst_v], add=True)
        if count:
            lane = lax.iota(jnp.int32, 16)
            ones16 = jnp.ones((16,), jnp.float32)
            for q in range(EB // 16):
                dq = dst_v[pl.ds(q * 16, 16)]
                for l in range(16):
                    plsc.addupdate_scatter(hist_v, [dq], ones16,
                                           mask=lane == l)
        return 0
    lax.fori_loop(0, NB, _step, 0)

    plsc.subcore_barrier()

    def _out(k, _):
        pltpu.sync_copy(acc.at[pl.ds(r0 + k * EB, EB)], rows_v)

        @pl.when(c == 0)
        def _():
            pltpu.sync_copy(rows_v, agg0.at[pl.ds(r0 + k * EB, EB)])

        @pl.when(c == 1)
        def _():
            pltpu.sync_copy(rows_v, agg1.at[pl.ds(r0 + k * EB, EB)])
        return 0
    lax.fori_loop(0, RPT // EB, _out, 0)

    if count:
        w = c * TILES + t
        pltpu.sync_copy(hist_v, cnt_out.at[w])


def _make_seg_sum(count):
    mesh = plsc.VectorSubcoreMesh(core_axis_name="c", subcore_axis_name="s")
    out_type = [jax.ShapeDtypeStruct((NPAD, DH), jnp.float32),
                jax.ShapeDtypeStruct((NPAD, DH), jnp.float32)]
    if count:
        out_type.append(jax.ShapeDtypeStruct((NW, NPAD), jnp.float32))
    return pl.kernel(
        functools.partial(_seg_sum_body, count),
        mesh=mesh,
        out_type=out_type,
        compiler_params=pltpu.CompilerParams(needs_layout_passes=False),
        scratch_types=[
            pltpu.VMEM((EB,), jnp.int32),
            pltpu.VMEM((EB,), jnp.int32),
            pltpu.VMEM((EB, DH), jnp.float32),
            pltpu.VMEM((NPAD,), jnp.float32),
            pltpu.VMEM_SHARED((NPAD, DH), jnp.float32),
            pltpu.SemaphoreType.DMA,
        ],
    )


def _recip_cnt(cnt_blk):
    ones_col = jnp.ones((NW, 1), jnp.float32)
    tot = lax.dot_general(cnt_blk, ones_col, (((0,), (0,)), ((), ())),
                          preferred_element_type=jnp.float32)
    return 1.0 / jnp.maximum(tot * 0.5, 1.0)


def _dense1_body(x, a0, a1, cnt, wla, wlb, wr, b, h0, h1):
    recip = _recip_cnt(cnt[...])
    m0 = a0[...] * recip
    m1 = a1[...] * recip
    dn = (((1,), (1,)), ((), ()))
    z = (lax.dot_general(m0, wla[...], dn, preferred_element_type=jnp.float32)
         + lax.dot_general(m1, wlb[...], dn, preferred_element_type=jnp.float32)
         + lax.dot_general(x[...], wr[...], dn, preferred_element_type=jnp.float32)
         + b[0:1, :])
    h = jnp.maximum(z, 0.0)
    h0[...] = h[:, :DH]
    h1[...] = h[:, DH:]


def _dense2_body(hh0, hh1, a0, a1, cnt, wla, wlb, wra, wrb, b, out):
    recip = _recip_cnt(cnt[...])
    m0 = a0[...] * recip
    m1 = a1[...] * recip
    dn = (((1,), (1,)), ((), ()))
    z = (lax.dot_general(m0, wla[...], dn, preferred_element_type=jnp.float32)
         + lax.dot_general(m1, wlb[...], dn, preferred_element_type=jnp.float32)
         + lax.dot_general(hh0[...], wra[...], dn, preferred_element_type=jnp.float32)
         + lax.dot_general(hh1[...], wrb[...], dn, preferred_element_type=jnp.float32)
         + b[0:1, :])
    m = jnp.max(z, axis=1, keepdims=True)
    lse = jnp.log(jnp.sum(jnp.exp(z - m), axis=1, keepdims=True)) + m
    out[...] = z - lse


def _row_spec(w):
    return pl.BlockSpec((RB, w), lambda i: (i, 0))


def _cnt_spec():
    return pl.BlockSpec((NW, RB), lambda i: (0, i))


def _full_spec(h, w):
    return pl.BlockSpec((h, w), lambda i: (0, 0))


_dense1 = pl.pallas_call(
    _dense1_body,
    grid=(GRID,),
    in_specs=[_row_spec(D), _row_spec(DH), _row_spec(DH), _cnt_spec(),
              _full_spec(D, DH), _full_spec(D, DH), _full_spec(D, D),
              _full_spec(8, D)],
    out_specs=[_row_spec(DH), _row_spec(DH)],
    out_shape=[jax.ShapeDtypeStruct((NPAD, DH), jnp.float32),
               jax.ShapeDtypeStruct((NPAD, DH), jnp.float32)],
)

_dense2 = pl.pallas_call(
    _dense2_body,
    grid=(GRID,),
    in_specs=[_row_spec(DH), _row_spec(DH), _row_spec(DH), _row_spec(DH),
              _cnt_spec(), _full_spec(D, DH), _full_spec(D, DH),
              _full_spec(D, DH), _full_spec(D, DH), _full_spec(8, D)],
    out_specs=_row_spec(D),
    out_shape=jax.ShapeDtypeStruct((NPAD, D), jnp.float32),
)


def kernel(x, edge_index, W_l1, b_l1, W_r1, W_l2, b_l2, W_r2):
    src = edge_index[0].astype(jnp.int32)
    dst = edge_index[1].astype(jnp.int32)
    x0 = x[:, :DH]
    x1 = x[:, DH:]
    xp = jnp.concatenate([x, jnp.zeros((NPAD - N, D), jnp.float32)], axis=0)
    b1 = jnp.tile(b_l1[None, :], (8, 1))
    b2 = jnp.tile(b_l2[None, :], (8, 1))

    agg0, agg1, cnt = _make_seg_sum(True)(x0, x1, src, dst)
    h0, h1 = _dense1(xp, agg0, agg1, cnt,
                     W_l1[:, :DH], W_l1[:, DH:], W_r1, b1)
    g0, g1 = _make_seg_sum(False)(h0, h1, src, dst)
    out = _dense2(h0, h1, g0, g1, cnt,
                  W_l2[:, :DH], W_l2[:, DH:], W_r2[:, :DH], W_r2[:, DH:], b2)
    return out[:N]


# speedup vs baseline: 6.0006x; 1.5815x over previous
import functools

import jax
import jax.numpy as jnp
from jax import lax
from jax.experimental import pallas as pl
from jax.experimental.pallas import tpu as pltpu
from jax.experimental.pallas import tpu_sc as plsc

N = 10000
NPAD = 10240
E = 160000
D = 256
DH = 128
EB = 48
NSUB = 5
GEB = EB * NSUB
TILES = 16
NW = 32
NG = 42
EPT = NG * GEB
EP = TILES * EPT
RPT = NPAD // TILES
ZB = 40
RB = 1024
GRID = NPAD // RB


def _seg_sum_body(count, *refs):
    if count:
        (x01, srch, dsth, agg01, cnt_out, *rest) = refs
    else:
        (x01, srch, dsth, agg01, *rest) = refs
        cnt_out = None
    src_v = rest[0:NSUB]
    dst_v = rest[NSUB:2 * NSUB]
    rows_v = rest[2 * NSUB:3 * NSUB]
    hist_v, acc, sem_i, sem_g, sem_s = rest[3 * NSUB:]
    c = lax.axis_index("c")
    t = lax.axis_index("s")
    r0 = t * RPT

    def _zero_rows(k, _):
        i = k // (DH // 16)
        j = k % (DH // 16)
        rows_v[0][i, pl.ds(j * 16, 16)] = jnp.zeros((16,), jnp.float32)
        return 0
    lax.fori_loop(0, EB * (DH // 16), _zero_rows, 0)

    def _zero_acc(k, _):
        pltpu.sync_copy(rows_v[0].at[pl.ds(0, ZB)],
                        acc.at[pl.ds(r0 + k * ZB, ZB)])
        return 0
    lax.fori_loop(0, RPT // ZB, _zero_acc, 0)

    if count:
        def _zero_hist(k, _):
            hist_v[pl.ds(k * 16, 16)] = jnp.zeros((16,), jnp.float32)
            return 0
        lax.fori_loop(0, NPAD // 16, _zero_hist, 0)

    plsc.subcore_barrier()

    ebase = t * EPT

    def _step(g, _):
        base = ebase + g * GEB
        ih = []
        for i in range(NSUB):
            ih.append(pltpu.async_copy(
                srch.at[pl.ds(base + i * EB, EB)], src_v[i], sem_i))
            ih.append(pltpu.async_copy(
                dsth.at[pl.ds(base + i * EB, EB)], dst_v[i], sem_i))
        for h in ih:
            h.wait()

        offv = jnp.full((16,), c * N, jnp.int32)
        for i in range(NSUB):
            for q in range(EB // 16):
                src_v[i][pl.ds(q * 16, 16)] = (
                    src_v[i][pl.ds(q * 16, 16)] + offv)

        gh = [pltpu.async_copy(x01.at[src_v[i]], rows_v[i], sem_g)
              for i in range(NSUB)]

        if count:
            lane = lax.iota(jnp.int32, 16)
            ones16 = jnp.ones((16,), jnp.float32)
            for i in range(NSUB):
                for q in range(EB // 16):
                    dq = dst_v[i][pl.ds(q * 16, 16)]
                    for l in range(16):
                        plsc.addupdate_scatter(hist_v, [dq], ones16,
                                               mask=lane == l)
        for h in gh:
            h.wait()

        sh = [pltpu.async_copy(rows_v[i], acc.at[dst_v[i]], sem_s, add=True)
              for i in range(NSUB)]
        for h in sh:
            h.wait()
        return 0
    lax.fori_loop(0, NG, _step, 0)

    plsc.subcore_barrier()

    def _out(k, _):
        pltpu.sync_copy(acc.at[pl.ds(r0 + k * ZB, ZB)],
                        rows_v[0].at[pl.ds(0, ZB)])
        pltpu.sync_copy(rows_v[0].at[pl.ds(0, ZB)],
                        agg01.at[pl.ds(c * NPAD + r0 + k * ZB, ZB)])
        return 0
    lax.fori_loop(0, RPT // ZB, _out, 0)

    if count:
        w = c * TILES + t
        pltpu.sync_copy(hist_v, cnt_out.at[w])


def _make_seg_sum(count):
    mesh = plsc.VectorSubcoreMesh(core_axis_name="c", subcore_axis_name="s")
    out_type = [jax.ShapeDtypeStruct((2 * NPAD, DH), jnp.float32)]
    if count:
        out_type.append(jax.ShapeDtypeStruct((NW, NPAD), jnp.float32))
    return pl.kernel(
        functools.partial(_seg_sum_body, count),
        mesh=mesh,
        out_type=out_type,
        compiler_params=pltpu.CompilerParams(needs_layout_passes=False),
        scratch_types=(
            [pltpu.VMEM((EB,), jnp.int32) for _ in range(2 * NSUB)]
            + [pltpu.VMEM((EB, DH), jnp.float32) for _ in range(NSUB)]
            + [pltpu.VMEM((NPAD,), jnp.float32),
               pltpu.VMEM_SHARED((NPAD, DH), jnp.float32),
               pltpu.SemaphoreType.DMA,
               pltpu.SemaphoreType.DMA,
               pltpu.SemaphoreType.DMA]
        ),
    )


def _recip_cnt(cnt_blk):
    ones_col = jnp.ones((NW, 1), jnp.float32)
    tot = lax.dot_general(cnt_blk, ones_col, (((0,), (0,)), ((), ())),
                          preferred_element_type=jnp.float32)
    return 1.0 / jnp.maximum(tot * 0.5, 1.0)


def _dense1_body(x, a0, a1, cnt, wla, wlb, wr, b, h0, h1):
    recip = _recip_cnt(cnt[...])
    m0 = a0[...] * recip
    m1 = a1[...] * recip
    dn = (((1,), (1,)), ((), ()))
    z = (lax.dot_general(m0, wla[...], dn, preferred_element_type=jnp.float32)
         + lax.dot_general(m1, wlb[...], dn, preferred_element_type=jnp.float32)
         + lax.dot_general(x[...], wr[...], dn, preferred_element_type=jnp.float32)
         + b[0:1, :])
    h = jnp.maximum(z, 0.0)
    h0[...] = h[:, :DH]
    h1[...] = h[:, DH:]


def _dense2_body(hh0, hh1, a0, a1, cnt, wla, wlb, wra, wrb, b, out):
    recip = _recip_cnt(cnt[...])
    m0 = a0[...] * recip
    m1 = a1[...] * recip
    dn = (((1,), (1,)), ((), ()))
    z = (lax.dot_general(m0, wla[...], dn, preferred_element_type=jnp.float32)
         + lax.dot_general(m1, wlb[...], dn, preferred_element_type=jnp.float32)
         + lax.dot_general(hh0[...], wra[...], dn, preferred_element_type=jnp.float32)
         + lax.dot_general(hh1[...], wrb[...], dn, preferred_element_type=jnp.float32)
         + b[0:1, :])
    m = jnp.max(z, axis=1, keepdims=True)
    lse = jnp.log(jnp.sum(jnp.exp(z - m), axis=1, keepdims=True)) + m
    out[...] = z - lse


def _row_spec(w):
    return pl.BlockSpec((RB, w), lambda i: (i, 0))


def _half_spec(half):
    if half == 0:
        return pl.BlockSpec((RB, DH), lambda i: (i, 0))
    return pl.BlockSpec((RB, DH), lambda i: (NPAD // RB + i, 0))


def _cnt_spec():
    return pl.BlockSpec((NW, RB), lambda i: (0, i))


def _full_spec(h, w):
    return pl.BlockSpec((h, w), lambda i: (0, 0))


_dense1 = pl.pallas_call(
    _dense1_body,
    grid=(GRID,),
    in_specs=[_row_spec(D), _half_spec(0), _half_spec(1), _cnt_spec(),
              _full_spec(D, DH), _full_spec(D, DH), _full_spec(D, D),
              _full_spec(8, D)],
    out_specs=[_row_spec(DH), _row_spec(DH)],
    out_shape=[jax.ShapeDtypeStruct((NPAD, DH), jnp.float32),
               jax.ShapeDtypeStruct((NPAD, DH), jnp.float32)],
)

_dense2 = pl.pallas_call(
    _dense2_body,
    grid=(GRID,),
    in_specs=[_row_spec(DH), _row_spec(DH), _half_spec(0), _half_spec(1),
              _cnt_spec(), _full_spec(D, DH), _full_spec(D, DH),
              _full_spec(D, DH), _full_spec(D, DH), _full_spec(8, D)],
    out_specs=_row_spec(D),
    out_shape=jax.ShapeDtypeStruct((NPAD, D), jnp.float32),
)


def kernel(x, edge_index, W_l1, b_l1, W_r1, W_l2, b_l2, W_r2):
    src = edge_index[0].astype(jnp.int32)
    dst = edge_index[1].astype(jnp.int32)
    npd = EP - E
    src = jnp.concatenate([src, jnp.arange(npd, dtype=jnp.int32) % N])
    dst = jnp.concatenate(
        [dst, N + (jnp.arange(npd, dtype=jnp.int32) % (NPAD - N))])
    x01 = jnp.concatenate([x[:, :DH], x[:, DH:]], axis=0)
    xp = jnp.concatenate([x, jnp.zeros((NPAD - N, D), jnp.float32)], axis=0)
    b1 = jnp.tile(b_l1[None, :], (8, 1))
    b2 = jnp.tile(b_l2[None, :], (8, 1))

    agg, cnt = _make_seg_sum(True)(x01, src, dst)
    h0, h1 = _dense1(xp, agg, agg, cnt,
                     W_l1[:, :DH], W_l1[:, DH:], W_r1, b1)
    h01 = jnp.concatenate([h0[:N], h1[:N]], axis=0)
    (agg2,) = _make_seg_sum(False)(h01, src, dst)
    out = _dense2(h0, h1, agg2, agg2, cnt,
                  W_l2[:, :DH], W_l2[:, DH:], W_r2[:, :DH], W_r2[:, DH:], b2)
    return out[:N]


# speedup vs baseline: 6.2852x; 1.0474x over previous
import functools

import jax
import jax.numpy as jnp
from jax import lax
from jax.experimental import pallas as pl
from jax.experimental.pallas import tpu as pltpu
from jax.experimental.pallas import tpu_sc as plsc

N = 10000
NPAD = 10240
E = 160000
D = 256
DH = 128
EB = 128
TILES = 16
NW = 32
EPT = 10240
EP = TILES * EPT
TROW = EPT // EB
SBR = 10
NSB = TROW // SBR
RPT = NPAD // TILES
ZB = 128
RB = 1024
GRID = NPAD // RB


def _seg_sum_body(count, *refs):
    if count:
        (x01, srch, dsth, agg01, cnt_out, srcb, *rest) = refs
    else:
        (x01, srch, dsth, agg01, srcb, *rest) = refs
        cnt_out = None
    dstb = rest[:SBR]
    row_a, row_b, hist_v, acc, sem_i, sem_g, sem_s = rest[SBR:]
    c = lax.axis_index("c")
    t = lax.axis_index("s")
    r0 = t * RPT
    rows = (row_a, row_b)

    def _zero_rows(k, _):
        i = k // (DH // 16)
        j = k % (DH // 16)
        row_a[i, pl.ds(j * 16, 16)] = jnp.zeros((16,), jnp.float32)
        return 0
    lax.fori_loop(0, ZB * (DH // 16), _zero_rows, 0)

    def _zero_acc(k, _):
        pltpu.sync_copy(row_a, acc.at[pl.ds(r0 + k * ZB, ZB)])
        return 0
    lax.fori_loop(0, RPT // ZB, _zero_acc, 0)

    if count:
        def _zero_hist(k, _):
            hist_v[pl.ds(k * 16, 16)] = jnp.zeros((16,), jnp.float32)
            return 0
        lax.fori_loop(0, NPAD // 16, _zero_hist, 0)

    plsc.subcore_barrier()

    def _hist_row(j):
        lane = lax.iota(jnp.int32, 16)
        ones16 = jnp.ones((16,), jnp.float32)
        for q in range(EB // 16):
            dq = dstb[j][pl.ds(q * 16, 16)]
            for l in range(16):
                plsc.addupdate_scatter(hist_v, [dq], ones16, mask=lane == l)

    trow = t * TROW

    def _sblock(m, _):
        ebase = (trow + m * SBR) * EB
        pltpu.sync_copy(srch.at[pl.ds(ebase, SBR * EB)], srcb)
        ih = [pltpu.async_copy(dsth.at[pl.ds(ebase + j * EB, EB)],
                               dstb[j], sem_i) for j in range(SBR)]

        offv = jnp.full((16,), c * N, jnp.int32)

        def _adj(k, _):
            srcb[pl.ds(k * 16, 16)] = srcb[pl.ds(k * 16, 16)] + offv
            return 0
        lax.fori_loop(0, SBR * (EB // 16), _adj, 0)
        for h in ih:
            h.wait()

        for it in range(SBR // 2):
            j0, j1 = 2 * it, 2 * it + 1
            g0 = pltpu.async_copy(x01.at[srcb.at[pl.ds(j0 * EB, EB)]],
                                  row_a, sem_g)
            if count:
                _hist_row(j0)
            g0.wait()
            s0 = pltpu.async_copy(row_a, acc.at[dstb[j0]], sem_s, add=True)
            g1 = pltpu.async_copy(x01.at[srcb.at[pl.ds(j1 * EB, EB)]],
                                  row_b, sem_g)
            if count:
                _hist_row(j1)
            g1.wait()
            s1 = pltpu.async_copy(row_b, acc.at[dstb[j1]], sem_s, add=True)
            s0.wait()
            s1.wait()
        return 0
    lax.fori_loop(0, NSB, _sblock, 0)

    plsc.subcore_barrier()

    def _out(k, _):
        pltpu.sync_copy(acc.at[pl.ds(r0 + k * ZB, ZB)], row_a)
        pltpu.sync_copy(row_a, agg01.at[pl.ds(c * NPAD + r0 + k * ZB, ZB)])
        return 0
    lax.fori_loop(0, RPT // ZB, _out, 0)

    if count:
        w = c * TILES + t
        pltpu.sync_copy(hist_v, cnt_out.at[w])


def _make_seg_sum(count):
    mesh = plsc.VectorSubcoreMesh(core_axis_name="c", subcore_axis_name="s")
    out_type = [jax.ShapeDtypeStruct((2 * NPAD, DH), jnp.float32)]
    if count:
        out_type.append(jax.ShapeDtypeStruct((NW, NPAD), jnp.float32))
    return pl.kernel(
        functools.partial(_seg_sum_body, count),
        mesh=mesh,
        out_type=out_type,
        compiler_params=pltpu.CompilerParams(needs_layout_passes=False),
        scratch_types=(
            [pltpu.VMEM((SBR * EB,), jnp.int32)]
            + [pltpu.VMEM((EB,), jnp.int32) for _ in range(SBR)]
            + [pltpu.VMEM((EB, DH), jnp.float32),
               pltpu.VMEM((EB, DH), jnp.float32),
               pltpu.VMEM((NPAD,), jnp.float32),
               pltpu.VMEM_SHARED((NPAD, DH), jnp.float32),
               pltpu.SemaphoreType.DMA,
               pltpu.SemaphoreType.DMA,
               pltpu.SemaphoreType.DMA]
        ),
    )


def _recip_cnt(cnt_blk):
    ones_col = jnp.ones((NW, 1), jnp.float32)
    tot = lax.dot_general(cnt_blk, ones_col, (((0,), (0,)), ((), ())),
                          preferred_element_type=jnp.float32)
    return 1.0 / jnp.maximum(tot * 0.5, 1.0)


def _dense1_body(x, a0, a1, cnt, wla, wlb, wr, b, h0, h1):
    recip = _recip_cnt(cnt[...])
    m0 = a0[...] * recip
    m1 = a1[...] * recip
    dn = (((1,), (1,)), ((), ()))
    z = (lax.dot_general(m0, wla[...], dn, preferred_element_type=jnp.float32)
         + lax.dot_general(m1, wlb[...], dn, preferred_element_type=jnp.float32)
         + lax.dot_general(x[...], wr[...], dn, preferred_element_type=jnp.float32)
         + b[0:1, :])
    h = jnp.maximum(z, 0.0)
    h0[...] = h[:, :DH]
    h1[...] = h[:, DH:]


def _dense2_body(hh0, hh1, a0, a1, cnt, wla, wlb, wra, wrb, b, out):
    recip = _recip_cnt(cnt[...])
    m0 = a0[...] * recip
    m1 = a1[...] * recip
    dn = (((1,), (1,)), ((), ()))
    z = (lax.dot_general(m0, wla[...], dn, preferred_element_type=jnp.float32)
         + lax.dot_general(m1, wlb[...], dn, preferred_element_type=jnp.float32)
         + lax.dot_general(hh0[...], wra[...], dn, preferred_element_type=jnp.float32)
         + lax.dot_general(hh1[...], wrb[...], dn, preferred_element_type=jnp.float32)
         + b[0:1, :])
    m = jnp.max(z, axis=1, keepdims=True)
    lse = jnp.log(jnp.sum(jnp.exp(z - m), axis=1, keepdims=True)) + m
    out[...] = z - lse


def _row_spec(w):
    return pl.BlockSpec((RB, w), lambda i: (i, 0))


def _half_spec(half):
    if half == 0:
        return pl.BlockSpec((RB, DH), lambda i: (i, 0))
    return pl.BlockSpec((RB, DH), lambda i: (NPAD // RB + i, 0))


def _cnt_spec():
    return pl.BlockSpec((NW, RB), lambda i: (0, i))


def _full_spec(h, w):
    return pl.BlockSpec((h, w), lambda i: (0, 0))


_dense1 = pl.pallas_call(
    _dense1_body,
    grid=(GRID,),
    in_specs=[_row_spec(D), _half_spec(0), _half_spec(1), _cnt_spec(),
              _full_spec(D, DH), _full_spec(D, DH), _full_spec(D, D),
              _full_spec(8, D)],
    out_specs=[_row_spec(DH), _row_spec(DH)],
    out_shape=[jax.ShapeDtypeStruct((NPAD, DH), jnp.float32),
               jax.ShapeDtypeStruct((NPAD, DH), jnp.float32)],
)

_dense2 = pl.pallas_call(
    _dense2_body,
    grid=(GRID,),
    in_specs=[_row_spec(DH), _row_spec(DH), _half_spec(0), _half_spec(1),
              _cnt_spec(), _full_spec(D, DH), _full_spec(D, DH),
              _full_spec(D, DH), _full_spec(D, DH), _full_spec(8, D)],
    out_specs=_row_spec(D),
    out_shape=jax.ShapeDtypeStruct((NPAD, D), jnp.float32),
)


def kernel(x, edge_index, W_l1, b_l1, W_r1, W_l2, b_l2, W_r2):
    src = edge_index[0].astype(jnp.int32)
    dst = edge_index[1].astype(jnp.int32)
    npd = EP - E
    src = jnp.concatenate([src, jnp.arange(npd, dtype=jnp.int32) % N])
    dst = jnp.concatenate(
        [dst, N + (jnp.arange(npd, dtype=jnp.int32) % (NPAD - N))])
    x01 = jnp.concatenate([x[:, :DH], x[:, DH:]], axis=0)
    xp = jnp.concatenate([x, jnp.zeros((NPAD - N, D), jnp.float32)], axis=0)
    b1 = jnp.tile(b_l1[None, :], (8, 1))
    b2 = jnp.tile(b_l2[None, :], (8, 1))

    agg, cnt = _make_seg_sum(True)(x01, src, dst)
    h0, h1 = _dense1(xp, agg, agg, cnt,
                     W_l1[:, :DH], W_l1[:, DH:], W_r1, b1)
    h01 = jnp.concatenate([h0[:N], h1[:N]], axis=0)
    (agg2,) = _make_seg_sum(False)(h01, src, dst)
    out = _dense2(h0, h1, agg2, agg2, cnt,
                  W_l2[:, :DH], W_l2[:, DH:], W_r2[:, :DH], W_r2[:, DH:], b2)
    return out[:N]


# speedup vs baseline: 7.2700x; 1.1567x over previous
import functools

import jax
import jax.numpy as jnp
from jax import lax
from jax.experimental import pallas as pl
from jax.experimental.pallas import tpu as pltpu
from jax.experimental.pallas import tpu_sc as plsc

N = 10000
NPAD = 10240
E = 160000
D = 256
DH = 128
EB = 128
TILES = 16
NW = 32
EPT = 10240
EP = TILES * EPT
TROW = EPT // EB
SBR = 10
NSB = TROW // SBR
RPT = NPAD // TILES
ZB = 128
RB = 1024
GRID = NPAD // RB


def _seg_sum_body(count, *refs):
    if count:
        (x01, srch, dsth, agg01, cnt_out, srcb, *rest) = refs
    else:
        (x01, srch, dsth, agg01, srcb, *rest) = refs
        cnt_out = None
    dstb = rest[:SBR]
    row_a, row_b, hist_v, acc, sem_i, sem_g, sem_s = rest[SBR:]
    c = lax.axis_index("c")
    t = lax.axis_index("s")
    r0 = t * RPT
    rows = (row_a, row_b)

    def _zero_rows(k, _):
        i = k // (DH // 16)
        j = k % (DH // 16)
        row_a[i, pl.ds(j * 16, 16)] = jnp.zeros((16,), jnp.float32)
        return 0
    lax.fori_loop(0, ZB * (DH // 16), _zero_rows, 0)

    def _zero_acc(k, _):
        pltpu.sync_copy(row_a, acc.at[pl.ds(r0 + k * ZB, ZB)])
        return 0
    lax.fori_loop(0, RPT // ZB, _zero_acc, 0)

    if count:
        def _zero_hist(k, _):
            hist_v[pl.ds(k * 16, 16)] = jnp.zeros((16,), jnp.float32)
            return 0
        lax.fori_loop(0, NPAD // 16, _zero_hist, 0)

    plsc.subcore_barrier()

    def _hist_row(j):
        lane = lax.iota(jnp.int32, 16)
        ones16 = jnp.ones((16,), jnp.float32)
        for q in range(EB // 16):
            dq = dstb[j][pl.ds(q * 16, 16)]
            for l in range(16):
                plsc.addupdate_scatter(hist_v, [dq], ones16, mask=lane == l)

    trow = t * TROW

    def _sblock(m, _):
        ebase = (trow + m * SBR) * EB
        pltpu.sync_copy(srch.at[pl.ds(ebase, SBR * EB)], srcb)
        ih = [pltpu.async_copy(dsth.at[pl.ds(ebase + j * EB, EB)],
                               dstb[j], sem_i) for j in range(SBR)]

        offv = jnp.full((16,), c * N, jnp.int32)

        def _adj(k, _):
            srcb[pl.ds(k * 16, 16)] = srcb[pl.ds(k * 16, 16)] + offv
            return 0
        lax.fori_loop(0, SBR * (EB // 16), _adj, 0)
        for h in ih:
            h.wait()

        for it in range(SBR // 2):
            j0, j1 = 2 * it, 2 * it + 1
            g0 = pltpu.async_copy(x01.at[srcb.at[pl.ds(j0 * EB, EB)]],
                                  row_a, sem_g)
            g1 = pltpu.async_copy(x01.at[srcb.at[pl.ds(j1 * EB, EB)]],
                                  row_b, sem_g)
            if count:
                _hist_row(j0)
                _hist_row(j1)
            g0.wait()
            s0 = pltpu.async_copy(row_a, acc.at[dstb[j0]], sem_s, add=True)
            g1.wait()
            s1 = pltpu.async_copy(row_b, acc.at[dstb[j1]], sem_s, add=True)
            s0.wait()
            s1.wait()
        return 0
    lax.fori_loop(0, NSB, _sblock, 0)

    plsc.subcore_barrier()

    def _out(k, _):
        pltpu.sync_copy(acc.at[pl.ds(r0 + k * ZB, ZB)], row_a)
        pltpu.sync_copy(row_a, agg01.at[pl.ds(c * NPAD + r0 + k * ZB, ZB)])
        return 0
    lax.fori_loop(0, RPT // ZB, _out, 0)

    if count:
        w = c * TILES + t
        pltpu.sync_copy(hist_v, cnt_out.at[w])


def _make_seg_sum(count):
    mesh = plsc.VectorSubcoreMesh(core_axis_name="c", subcore_axis_name="s")
    out_type = [jax.ShapeDtypeStruct((2 * NPAD, DH), jnp.float32)]
    if count:
        out_type.append(jax.ShapeDtypeStruct((NW, NPAD), jnp.float32))
    return pl.kernel(
        functools.partial(_seg_sum_body, count),
        mesh=mesh,
        out_type=out_type,
        compiler_params=pltpu.CompilerParams(needs_layout_passes=False),
        scratch_types=(
            [pltpu.VMEM((SBR * EB,), jnp.int32)]
            + [pltpu.VMEM((EB,), jnp.int32) for _ in range(SBR)]
            + [pltpu.VMEM((EB, DH), jnp.float32),
               pltpu.VMEM((EB, DH), jnp.float32),
               pltpu.VMEM((NPAD,), jnp.float32),
               pltpu.VMEM_SHARED((NPAD, DH), jnp.float32),
               pltpu.SemaphoreType.DMA,
               pltpu.SemaphoreType.DMA,
               pltpu.SemaphoreType.DMA]
        ),
    )


def _recip_cnt(cnt_blk):
    ones_col = jnp.ones((NW, 1), jnp.float32)
    tot = lax.dot_general(cnt_blk, ones_col, (((0,), (0,)), ((), ())),
                          preferred_element_type=jnp.float32)
    return 1.0 / jnp.maximum(tot * 0.5, 1.0)


def _pre1_body(x, wr, b, o):
    dn = (((1,), (1,)), ((), ()))
    o[...] = (lax.dot_general(x[...], wr[...], dn,
                              preferred_element_type=jnp.float32)
              + b[0:1, :])


def _pre2_body(hh0, hh1, wra, wrb, b, o):
    dn = (((1,), (1,)), ((), ()))
    o[...] = (lax.dot_general(hh0[...], wra[...], dn,
                              preferred_element_type=jnp.float32)
              + lax.dot_general(hh1[...], wrb[...], dn,
                                preferred_element_type=jnp.float32)
              + b[0:1, :])


def _dense1_body(pre, a0, a1, cnt, wla, wlb, h0, h1):
    recip = _recip_cnt(cnt[...])
    m0 = a0[...] * recip
    m1 = a1[...] * recip
    dn = (((1,), (1,)), ((), ()))
    z = (lax.dot_general(m0, wla[...], dn, preferred_element_type=jnp.float32)
         + lax.dot_general(m1, wlb[...], dn, preferred_element_type=jnp.float32)
         + pre[...])
    h = jnp.maximum(z, 0.0)
    h0[...] = h[:, :DH]
    h1[...] = h[:, DH:]


def _dense2_body(pre, a0, a1, cnt, wla, wlb, out):
    recip = _recip_cnt(cnt[...])
    m0 = a0[...] * recip
    m1 = a1[...] * recip
    dn = (((1,), (1,)), ((), ()))
    z = (lax.dot_general(m0, wla[...], dn, preferred_element_type=jnp.float32)
         + lax.dot_general(m1, wlb[...], dn, preferred_element_type=jnp.float32)
         + pre[...])
    m = jnp.max(z, axis=1, keepdims=True)
    lse = jnp.log(jnp.sum(jnp.exp(z - m), axis=1, keepdims=True)) + m
    out[...] = z - lse


def _row_spec(w):
    return pl.BlockSpec((RB, w), lambda i: (i, 0))


def _half_spec(half):
    if half == 0:
        return pl.BlockSpec((RB, DH), lambda i: (i, 0))
    return pl.BlockSpec((RB, DH), lambda i: (NPAD // RB + i, 0))


def _cnt_spec():
    return pl.BlockSpec((NW, RB), lambda i: (0, i))


def _full_spec(h, w):
    return pl.BlockSpec((h, w), lambda i: (0, 0))


_pre1 = pl.pallas_call(
    _pre1_body,
    grid=(GRID,),
    in_specs=[_row_spec(D), _full_spec(D, D), _full_spec(8, D)],
    out_specs=_row_spec(D),
    out_shape=jax.ShapeDtypeStruct((N, D), jnp.float32),
)

_pre2 = pl.pallas_call(
    _pre2_body,
    grid=(GRID,),
    in_specs=[_row_spec(DH), _row_spec(DH), _full_spec(D, DH),
              _full_spec(D, DH), _full_spec(8, D)],
    out_specs=_row_spec(D),
    out_shape=jax.ShapeDtypeStruct((N, D), jnp.float32),
)

_dense1 = pl.pallas_call(
    _dense1_body,
    grid=(GRID,),
    in_specs=[_row_spec(D), _half_spec(0), _half_spec(1), _cnt_spec(),
              _full_spec(D, DH), _full_spec(D, DH)],
    out_specs=[_row_spec(DH), _row_spec(DH)],
    out_shape=[jax.ShapeDtypeStruct((N, DH), jnp.float32),
               jax.ShapeDtypeStruct((N, DH), jnp.float32)],
)

_dense2 = pl.pallas_call(
    _dense2_body,
    grid=(GRID,),
    in_specs=[_row_spec(D), _half_spec(0), _half_spec(1), _cnt_spec(),
              _full_spec(D, DH), _full_spec(D, DH)],
    out_specs=_row_spec(D),
    out_shape=jax.ShapeDtypeStruct((N, D), jnp.float32),
)


def kernel(x, edge_index, W_l1, b_l1, W_r1, W_l2, b_l2, W_r2):
    src = edge_index[0].astype(jnp.int32)
    dst = edge_index[1].astype(jnp.int32)
    npd = EP - E
    src = jnp.concatenate([src, jnp.arange(npd, dtype=jnp.int32) % N])
    dst = jnp.concatenate(
        [dst, N + (jnp.arange(npd, dtype=jnp.int32) % (NPAD - N))])
    x01 = jnp.concatenate([x[:, :DH], x[:, DH:]], axis=0)
    b1 = jnp.tile(b_l1[None, :], (8, 1))
    b2 = jnp.tile(b_l2[None, :], (8, 1))

    agg, cnt = _make_seg_sum(True)(x01, src, dst)
    pre1 = _pre1(x, W_r1, b1)
    h0, h1 = _dense1(pre1, agg, agg, cnt, W_l1[:, :DH], W_l1[:, DH:])
    h01 = jnp.concatenate([h0, h1], axis=0)
    (agg2,) = _make_seg_sum(False)(h01, src, dst)
    pre2 = _pre2(h0, h1, W_r2[:, :DH], W_r2[:, DH:], b2)
    out = _dense2(pre2, agg2, agg2, cnt, W_l2[:, :DH], W_l2[:, DH:])
    return out


# speedup vs baseline: 7.7500x; 1.0660x over previous
import functools

import jax
import jax.numpy as jnp
from jax import lax
from jax.experimental import pallas as pl
from jax.experimental.pallas import tpu as pltpu
from jax.experimental.pallas import tpu_sc as plsc

N = 10000
NPAD = 10240
E = 160000
D = 256
DH = 128
EB = 128
SB = 64
NBUF = 4
TILES = 16
NW = 32
EPT = 10240
EP = TILES * EPT
TROW = EPT // EB
SBR = 10
NSB = TROW // SBR
RPT = NPAD // TILES
ZB = 128
RB = 1024
GRID = NPAD // RB


def _seg_sum_body(count, *refs):
    if count:
        (x01, srch, dsth, agg01, cnt_out, srcb, *rest) = refs
    else:
        (x01, srch, dsth, agg01, srcb, *rest) = refs
        cnt_out = None
    nsb = SBR * EB // SB
    dstb = rest[:nsb]
    rows = rest[nsb:nsb + NBUF]
    hist_v, acc, sem_i, sem_g, sem_s = rest[nsb + NBUF:]
    row_a = rows[0]
    c = lax.axis_index("c")
    t = lax.axis_index("s")
    r0 = t * RPT

    def _zero_rows(k, _):
        i = k // (DH // 16)
        j = k % (DH // 16)
        row_a[i, pl.ds(j * 16, 16)] = jnp.zeros((16,), jnp.float32)
        return 0
    lax.fori_loop(0, SB * (DH // 16), _zero_rows, 0)

    zh = [pltpu.async_copy(row_a, acc.at[pl.ds(r0 + k * SB, SB)], sem_s)
          for k in range(RPT // SB)]
    for h in zh:
        h.wait()

    if count:
        def _zero_hist(k, _):
            hist_v[pl.ds(k * 16, 16)] = jnp.zeros((16,), jnp.float32)
            return 0
        lax.fori_loop(0, NPAD // 16, _zero_hist, 0)

    plsc.subcore_barrier()

    def _hist_row(j):
        lane = lax.iota(jnp.int32, 16)
        ones16 = jnp.ones((16,), jnp.float32)
        for q in range(SB // 16):
            dq = dstb[j][pl.ds(q * 16, 16)]
            for l in range(16):
                plsc.addupdate_scatter(hist_v, [dq], ones16, mask=lane == l)

    trow = t * TROW

    nsb = SBR * EB // SB

    def _sblock(m, _):
        ebase = (trow + m * SBR) * EB
        pltpu.sync_copy(srch.at[pl.ds(ebase, SBR * EB)], srcb)
        ih = [pltpu.async_copy(dsth.at[pl.ds(ebase + j * SB, SB)],
                               dstb[j], sem_i) for j in range(nsb)]

        offv = jnp.full((16,), c * N, jnp.int32)

        def _adj(k, _):
            srcb[pl.ds(k * 16, 16)] = srcb[pl.ds(k * 16, 16)] + offv
            return 0
        lax.fori_loop(0, SBR * (EB // 16), _adj, 0)
        for h in ih:
            h.wait()

        for it in range(nsb // NBUF):
            js = [NBUF * it + i for i in range(NBUF)]
            gs = [pltpu.async_copy(
                x01.at[srcb.at[pl.ds(j * SB, SB)]], rows[i], sem_g)
                for i, j in enumerate(js)]
            if count:
                for j in js:
                    _hist_row(j)
            ss = []
            for i, j in enumerate(js):
                gs[i].wait()
                ss.append(pltpu.async_copy(rows[i], acc.at[dstb[j]],
                                           sem_s, add=True))
            for s in ss:
                s.wait()
        return 0
    lax.fori_loop(0, NSB, _sblock, 0)

    plsc.subcore_barrier()

    oh = []
    for k in range(RPT // SB):
        b = k % NBUF
        if k >= NBUF:
            oh[k - NBUF].wait()
        pltpu.async_copy(acc.at[pl.ds(r0 + k * SB, SB)], rows[b],
                         sem_g).wait()
        oh.append(pltpu.async_copy(
            rows[b], agg01.at[pl.ds(c * NPAD + r0 + k * SB, SB)], sem_s))
    for k in range(RPT // SB - NBUF, RPT // SB):
        oh[k].wait()

    if count:
        w = c * TILES + t
        pltpu.sync_copy(hist_v, cnt_out.at[w])


def _make_seg_sum(count):
    mesh = plsc.VectorSubcoreMesh(core_axis_name="c", subcore_axis_name="s")
    out_type = [jax.ShapeDtypeStruct((2 * NPAD, DH), jnp.float32)]
    if count:
        out_type.append(jax.ShapeDtypeStruct((NW, NPAD), jnp.float32))
    return pl.kernel(
        functools.partial(_seg_sum_body, count),
        mesh=mesh,
        out_type=out_type,
        compiler_params=pltpu.CompilerParams(needs_layout_passes=False),
        scratch_types=(
            [pltpu.VMEM((SBR * EB,), jnp.int32)]
            + [pltpu.VMEM((SB,), jnp.int32) for _ in range(SBR * EB // SB)]
            + [pltpu.VMEM((SB, DH), jnp.float32) for _ in range(NBUF)]
            + [pltpu.VMEM((NPAD,), jnp.float32),
               pltpu.VMEM_SHARED((NPAD, DH), jnp.float32),
               pltpu.SemaphoreType.DMA,
               pltpu.SemaphoreType.DMA,
               pltpu.SemaphoreType.DMA]
        ),
    )


def _recip_cnt(cnt_blk):
    ones_col = jnp.ones((NW, 1), jnp.float32)
    tot = lax.dot_general(cnt_blk, ones_col, (((0,), (0,)), ((), ())),
                          preferred_element_type=jnp.float32)
    return 1.0 / jnp.maximum(tot * 0.5, 1.0)


def _pre1_body(x, wr, b, o):
    dn = (((1,), (1,)), ((), ()))
    o[...] = (lax.dot_general(x[...], wr[...], dn,
                              preferred_element_type=jnp.float32)
              + b[0:1, :])


def _pre2_body(hh0, hh1, wra, wrb, b, o):
    dn = (((1,), (1,)), ((), ()))
    o[...] = (lax.dot_general(hh0[...], wra[...], dn,
                              preferred_element_type=jnp.float32)
              + lax.dot_general(hh1[...], wrb[...], dn,
                                preferred_element_type=jnp.float32)
              + b[0:1, :])


def _dense1_body(pre, a0, a1, cnt, wla, wlb, h0, h1):
    recip = _recip_cnt(cnt[...])
    m0 = a0[...] * recip
    m1 = a1[...] * recip
    dn = (((1,), (1,)), ((), ()))
    z = (lax.dot_general(m0, wla[...], dn, preferred_element_type=jnp.float32)
         + lax.dot_general(m1, wlb[...], dn, preferred_element_type=jnp.float32)
         + pre[...])
    h = jnp.maximum(z, 0.0)
    h0[...] = h[:, :DH]
    h1[...] = h[:, DH:]


def _dense2_body(pre, a0, a1, cnt, wla, wlb, out):
    recip = _recip_cnt(cnt[...])
    m0 = a0[...] * recip
    m1 = a1[...] * recip
    dn = (((1,), (1,)), ((), ()))
    z = (lax.dot_general(m0, wla[...], dn, preferred_element_type=jnp.float32)
         + lax.dot_general(m1, wlb[...], dn, preferred_element_type=jnp.float32)
         + pre[...])
    m = jnp.max(z, axis=1, keepdims=True)
    lse = jnp.log(jnp.sum(jnp.exp(z - m), axis=1, keepdims=True)) + m
    out[...] = z - lse


def _row_spec(w):
    return pl.BlockSpec((RB, w), lambda i: (i, 0))


def _half_spec(half):
    if half == 0:
        return pl.BlockSpec((RB, DH), lambda i: (i, 0))
    return pl.BlockSpec((RB, DH), lambda i: (NPAD // RB + i, 0))


def _cnt_spec():
    return pl.BlockSpec((NW, RB), lambda i: (0, i))


def _full_spec(h, w):
    return pl.BlockSpec((h, w), lambda i: (0, 0))


_pre1 = pl.pallas_call(
    _pre1_body,
    grid=(GRID,),
    in_specs=[_row_spec(D), _full_spec(D, D), _full_spec(8, D)],
    out_specs=_row_spec(D),
    out_shape=jax.ShapeDtypeStruct((N, D), jnp.float32),
)

_pre2 = pl.pallas_call(
    _pre2_body,
    grid=(GRID,),
    in_specs=[_row_spec(DH), _row_spec(DH), _full_spec(D, DH),
              _full_spec(D, DH), _full_spec(8, D)],
    out_specs=_row_spec(D),
    out_shape=jax.ShapeDtypeStruct((N, D), jnp.float32),
)

_dense1 = pl.pallas_call(
    _dense1_body,
    grid=(GRID,),
    in_specs=[_row_spec(D), _half_spec(0), _half_spec(1), _cnt_spec(),
              _full_spec(D, DH), _full_spec(D, DH)],
    out_specs=[_row_spec(DH), _row_spec(DH)],
    out_shape=[jax.ShapeDtypeStruct((N, DH), jnp.float32),
               jax.ShapeDtypeStruct((N, DH), jnp.float32)],
)

_dense2 = pl.pallas_call(
    _dense2_body,
    grid=(GRID,),
    in_specs=[_row_spec(D), _half_spec(0), _half_spec(1), _cnt_spec(),
              _full_spec(D, DH), _full_spec(D, DH)],
    out_specs=_row_spec(D),
    out_shape=jax.ShapeDtypeStruct((N, D), jnp.float32),
)


def kernel(x, edge_index, W_l1, b_l1, W_r1, W_l2, b_l2, W_r2):
    src = edge_index[0].astype(jnp.int32)
    dst = edge_index[1].astype(jnp.int32)
    npd = EP - E
    src = jnp.concatenate([src, jnp.arange(npd, dtype=jnp.int32) % N])
    dst = jnp.concatenate(
        [dst, N + (jnp.arange(npd, dtype=jnp.int32) % (NPAD - N))])
    x01 = jnp.concatenate([x[:, :DH], x[:, DH:]], axis=0)
    b1 = jnp.tile(b_l1[None, :], (8, 1))
    b2 = jnp.tile(b_l2[None, :], (8, 1))

    agg, cnt = _make_seg_sum(True)(x01, src, dst)
    pre1 = _pre1(x, W_r1, b1)
    h0, h1 = _dense1(pre1, agg, agg, cnt, W_l1[:, :DH], W_l1[:, DH:])
    h01 = jnp.concatenate([h0, h1], axis=0)
    (agg2,) = _make_seg_sum(False)(h01, src, dst)
    pre2 = _pre2(h0, h1, W_r2[:, :DH], W_r2[:, DH:], b2)
    out = _dense2(pre2, agg2, agg2, cnt, W_l2[:, :DH], W_l2[:, DH:])
    return out
